# baseline (speedup 1.0000x reference)
import functools

import jax
import jax.numpy as jnp
from jax import lax
from jax.experimental import pallas as pl
from jax.experimental.pallas import tpu as pltpu
from jax.experimental.pallas import tpu_sc as plsc

BATCH = 4096
HIST = 200
F = 32
NC = 2
NS = 16
NW = NC * NS
BB = BATCH // NW
FH = F // 8
NBUF = 4
DEPTH = NBUF - 1
NSTEP = HIST // NBUF


def _fire_gather(tab_hbm, idx_all, rows, gsem, l):
    pltpu.async_copy(tab_hbm.at[idx_all.at[l]], rows, gsem)


def _wait_gather(tab_hbm, idx_all, rows, gsem):
    pltpu.make_async_copy(tab_hbm.at[idx_all.at[0]], rows, gsem).wait()


def _fire_wb(out_hbm, trows, wsem, l, w):
    for fh in range(FH):
        pltpu.async_copy(
            trows.at[pl.ds(fh * 8 * BB, 8 * BB)], out_hbm.at[l, fh, w], wsem)


def _wait_wb(out_hbm, trows, wsem, w):
    for fh in range(FH):
        pltpu.make_async_copy(
            trows.at[pl.ds(fh * 8 * BB, 8 * BB)], out_hbm.at[0, fh, w],
            wsem).wait()


def _mask_chunk(idx_all, rows, l):
    acc = jnp.zeros((16,), jnp.int32)
    for g in range(BB // 16):
        v = idx_all[l, pl.ds(g * 16, 16)]
        acc = acc + jnp.where(v <= 0, 1, 0).astype(jnp.int32)
    nz = plsc.all_reduce_population_count(acc > 0)[0]

    @pl.when(nz > 0)
    def _():
        zeros = jnp.zeros((16,), jnp.float32)

        def fix(g, carry):
            off = pl.multiple_of(g * 16, 16)
            v = idx_all[l, pl.ds(off, 16)]
            m = v <= 0
            rid = lax.iota(jnp.int32, 16) + g * 16
            for c in range(F):
                plsc.store_scatter(
                    rows, [rid, jnp.full((16,), c, jnp.int32)],
                    zeros, mask=m)
            return carry

        lax.fori_loop(0, BB // 16, fix, 0)


def _transpose_chunk(rows, trows, iota_bb):
    @plsc.parallel_loop(0, BB, 16, unroll=2)
    def _(b0):
        base0 = iota_bb + b0
        base1 = base0 + 16 * BB
        for j in range(16):
            for h, base in ((0, base0), (1, base1)):
                v = rows[b0 + j, pl.ds(h * 16, 16)]
                plsc.store_scatter(trows, [base + j], v)


@functools.partial(
    pl.kernel,
    out_type=jax.ShapeDtypeStruct((HIST, FH, NW, 8 * BB), jnp.float32),
    mesh=plsc.VectorSubcoreMesh(core_axis_name="c", subcore_axis_name="s"),
    compiler_params=pltpu.CompilerParams(
        needs_layout_passes=False, use_tc_tiling_on_sc=False),
    scratch_types=[
        pltpu.VMEM((HIST, BB), jnp.int32),
        [pltpu.VMEM((BB, F), jnp.float32) for _ in range(NBUF)],
        [pltpu.VMEM((F * BB,), jnp.float32) for _ in range(NBUF)],
        [pltpu.SemaphoreType.DMA for _ in range(NBUF)],
        [pltpu.SemaphoreType.DMA for _ in range(NBUF)],
    ],
)
def _embed(x_hbm, tab_hbm, out_hbm, idx_all, rows, trows, gsem, wsem):
    wid = lax.axis_index("s") * NC + lax.axis_index("c")
    iota_bb = lax.iota(jnp.int32, 16) * BB

    pltpu.sync_copy(x_hbm.at[:, pl.ds(wid * BB, BB)], idx_all)

    for l in range(DEPTH):
        _fire_gather(tab_hbm, idx_all, rows[l], gsem[l], l)

    def step(i, carry):
        for b in range(NBUF):
            l = i * NBUF + b
            _wait_gather(tab_hbm, idx_all, rows[b], gsem[b])
            _mask_chunk(idx_all, rows[b], l)
            @pl.when(i > 0)
            def _():
                _wait_wb(out_hbm, trows[b], wsem[b], wid)
            _transpose_chunk(rows[b], trows[b], iota_bb)
            _fire_wb(out_hbm, trows[b], wsem[b], l, wid)
            nb = (b + DEPTH) % NBUF
            if b == 0:
                _fire_gather(tab_hbm, idx_all, rows[nb], gsem[nb], l + DEPTH)
            else:
                @pl.when(i < NSTEP - 1)
                def _():
                    _fire_gather(tab_hbm, idx_all, rows[nb], gsem[nb],
                                 l + DEPTH)
        return carry

    lax.fori_loop(0, NSTEP, step, 0)

    for b in range(NBUF):
        _wait_wb(out_hbm, trows[b], wsem[b], wid)


def kernel(x, table):
    x2 = jnp.transpose(x.astype(jnp.int32)[:, :, 0], (1, 0))
    out4 = _embed(x2, table)
    out5 = out4.reshape(HIST, FH, NW, 8, BB)
    return out5.transpose(2, 4, 0, 1, 3).reshape(BATCH, HIST, F)


# speedup vs baseline: 1.3153x; 1.3153x over previous
import functools

import jax
import jax.numpy as jnp
from jax import lax
from jax.experimental import pallas as pl
from jax.experimental.pallas import tpu as pltpu
from jax.experimental.pallas import tpu_sc as plsc

BATCH = 4096
HIST = 200
F = 32
NC = 2
NS = 16
NW = NC * NS
BB = BATCH // NW
FH = F // 8
NBUF = 4
DEPTH = NBUF - 1
NSTEP = HIST // NBUF


def _fire_gather(tab_hbm, idx_all, rows, gsem, l):
    pltpu.async_copy(tab_hbm.at[idx_all.at[l]], rows, gsem)


def _wait_gather(tab_hbm, idx_all, rows, gsem):
    pltpu.make_async_copy(tab_hbm.at[idx_all.at[0]], rows, gsem).wait()


def _fire_wb(out_hbm, trows, wsem, l, w):
    for fh in range(FH):
        pltpu.async_copy(
            trows.at[pl.ds(fh * 8, 8), pl.ds(0, BB)],
            out_hbm.at[l, fh, w], wsem)


def _wait_wb(out_hbm, trows, wsem, w):
    for fh in range(FH):
        pltpu.make_async_copy(
            trows.at[pl.ds(fh * 8, 8), pl.ds(0, BB)], out_hbm.at[0, fh, w],
            wsem).wait()


def _mask_chunk(idx_all, rows, l):
    acc = jnp.zeros((16,), jnp.int32)
    for g in range(BB // 16):
        v = idx_all[l, pl.ds(g * 16, 16)]
        acc = acc + jnp.where(v <= 0, 1, 0).astype(jnp.int32)
    nz = plsc.all_reduce_population_count(acc > 0)[0]

    @pl.when(nz > 0)
    def _():
        zeros = jnp.zeros((16,), jnp.float32)

        def fix(g, carry):
            off = pl.multiple_of(g * 16, 16)
            v = idx_all[l, pl.ds(off, 16)]
            m = v <= 0
            rid = lax.iota(jnp.int32, 16) + g * 16
            for c in range(F):
                plsc.store_scatter(
                    rows, [rid, jnp.full((16,), c, jnp.int32)],
                    zeros, mask=m)
            return carry

        lax.fori_loop(0, BB // 16, fix, 0)


def _transpose_chunk(rows, trows, iota16):
    cvec = (iota16, iota16 + 16)

    @plsc.parallel_loop(0, BB, 16, unroll=2)
    def _(b0):
        for j in range(16):
            col = jnp.full((16,), 0, jnp.int32) + (b0 + j)
            for h in range(2):
                v = rows[b0 + j, pl.ds(h * 16, 16)]
                plsc.store_scatter(trows, [cvec[h], col], v)


@functools.partial(
    pl.kernel,
    out_type=jax.ShapeDtypeStruct((HIST, FH, NW, 8, BB), jnp.float32),
    mesh=plsc.VectorSubcoreMesh(core_axis_name="c", subcore_axis_name="s"),
    compiler_params=pltpu.CompilerParams(
        needs_layout_passes=False, use_tc_tiling_on_sc=False),
    scratch_types=[
        pltpu.VMEM((HIST, BB), jnp.int32),
        [pltpu.VMEM((BB, F), jnp.float32) for _ in range(NBUF)],
        [pltpu.VMEM((F, BB + 1), jnp.float32) for _ in range(NBUF)],
        [pltpu.SemaphoreType.DMA for _ in range(NBUF)],
        [pltpu.SemaphoreType.DMA for _ in range(NBUF)],
    ],
)
def _embed(x_hbm, tab_hbm, out_hbm, idx_all, rows, trows, gsem, wsem):
    wid = lax.axis_index("s") * NC + lax.axis_index("c")
    iota16 = lax.iota(jnp.int32, 16)

    pltpu.sync_copy(x_hbm.at[:, pl.ds(wid * BB, BB)], idx_all)

    for l in range(DEPTH):
        _fire_gather(tab_hbm, idx_all, rows[l], gsem[l], l)

    def step(i, carry):
        for b in range(NBUF):
            l = i * NBUF + b
            _wait_gather(tab_hbm, idx_all, rows[b], gsem[b])
            _mask_chunk(idx_all, rows[b], l)
            @pl.when(i > 0)
            def _():
                _wait_wb(out_hbm, trows[b], wsem[b], wid)
            _transpose_chunk(rows[b], trows[b], iota16)
            _fire_wb(out_hbm, trows[b], wsem[b], l, wid)
            nb = (b + DEPTH) % NBUF
            if b == 0:
                _fire_gather(tab_hbm, idx_all, rows[nb], gsem[nb], l + DEPTH)
            else:
                @pl.when(i < NSTEP - 1)
                def _():
                    _fire_gather(tab_hbm, idx_all, rows[nb], gsem[nb],
                                 l + DEPTH)
        return carry

    lax.fori_loop(0, NSTEP, step, 0)

    for b in range(NBUF):
        _wait_wb(out_hbm, trows[b], wsem[b], wid)


def kernel(x, table):
    x2 = jnp.transpose(x.astype(jnp.int32)[:, :, 0], (1, 0))
    out5 = _embed(x2, table)
    return out5.transpose(2, 4, 0, 1, 3).reshape(BATCH, HIST, F)


# speedup vs baseline: 1.3406x; 1.0193x over previous
import functools

import jax
import jax.numpy as jnp
from jax import lax
from jax.experimental import pallas as pl
from jax.experimental.pallas import tpu as pltpu
from jax.experimental.pallas import tpu_sc as plsc

BATCH = 4096
HIST = 200
F = 32
NC = 2
NS = 16
NW = NC * NS
BB = BATCH // NW
FH = F // 8
NBUF = 4
DEPTH = NBUF - 1
NSTEP = HIST // NBUF


def _fire_gather(tab_hbm, idx_all, rows, gsem, l):
    pltpu.async_copy(tab_hbm.at[idx_all.at[l]], rows, gsem)


def _wait_gather(tab_hbm, idx_all, rows, gsem):
    pltpu.make_async_copy(tab_hbm.at[idx_all.at[0]], rows, gsem).wait()


def _fire_wb(out_hbm, trows, wsem, l, w):
    for fh in range(FH):
        pltpu.async_copy(
            trows.at[pl.ds(fh * 8, 8), pl.ds(0, BB)],
            out_hbm.at[l, fh, w], wsem)


def _wait_wb(out_hbm, trows, wsem, w):
    for fh in range(FH):
        pltpu.make_async_copy(
            trows.at[pl.ds(fh * 8, 8), pl.ds(0, BB)], out_hbm.at[0, fh, w],
            wsem).wait()


def _mask_chunk(idx_all, rows, l):
    acc = jnp.zeros((16,), jnp.int32)
    for g in range(BB // 16):
        v = idx_all[l, pl.ds(g * 16, 16)]
        acc = acc + jnp.where(v <= 0, 1, 0).astype(jnp.int32)
    nz = plsc.all_reduce_population_count(acc > 0)[0]

    @pl.when(nz > 0)
    def _():
        zeros = jnp.zeros((16,), jnp.float32)

        def fix(g, carry):
            off = pl.multiple_of(g * 16, 16)
            v = idx_all[l, pl.ds(off, 16)]
            m = v <= 0
            rid = lax.iota(jnp.int32, 16) + g * 16
            for c in range(F):
                plsc.store_scatter(
                    rows, [rid, jnp.full((16,), c, jnp.int32)],
                    zeros, mask=m)
            return carry

        lax.fori_loop(0, BB // 16, fix, 0)


def _transpose_chunk(rows, trows, iota16):
    cvec = (iota16, iota16 + 16)

    @plsc.parallel_loop(0, BB, 16, unroll=2)
    def _(b0):
        for j in range(16):
            col = jnp.full((16,), 0, jnp.int32) + (b0 + j)
            for h in range(2):
                v = rows[b0 + j, pl.ds(h * 16, 16)]
                plsc.store_scatter(trows, [cvec[h], col], v)


@functools.partial(
    pl.kernel,
    out_type=jax.ShapeDtypeStruct((HIST, FH, NW, 8, BB), jnp.float32),
    mesh=plsc.VectorSubcoreMesh(core_axis_name="c", subcore_axis_name="s"),
    compiler_params=pltpu.CompilerParams(
        needs_layout_passes=False, use_tc_tiling_on_sc=False),
    scratch_types=[
        pltpu.VMEM((HIST, BB), jnp.int32),
        [pltpu.VMEM((BB, F), jnp.float32) for _ in range(NBUF)],
        [pltpu.VMEM((F, BB + 1), jnp.float32) for _ in range(NBUF)],
        [pltpu.SemaphoreType.DMA for _ in range(NBUF)],
        [pltpu.SemaphoreType.DMA for _ in range(NBUF)],
    ],
)
def _embed(x_hbm, tab_hbm, out_hbm, idx_all, rows, trows, gsem, wsem):
    wid = lax.axis_index("s") * NC + lax.axis_index("c")
    iota16 = lax.iota(jnp.int32, 16)

    pltpu.sync_copy(x_hbm.at[:, pl.ds(wid * BB, BB)], idx_all)

    for l in range(DEPTH):
        _fire_gather(tab_hbm, idx_all, rows[l], gsem[l], l)

    def step(i, carry):
        for b in range(NBUF):
            l = i * NBUF + b
            _wait_gather(tab_hbm, idx_all, rows[b], gsem[b])
            _mask_chunk(idx_all, rows[b], l)
            @pl.when(i > 0)
            def _():
                _wait_wb(out_hbm, trows[b], wsem[b], wid)
            _transpose_chunk(rows[b], trows[b], iota16)
            _fire_wb(out_hbm, trows[b], wsem[b], l, wid)
            nb = (b + DEPTH) % NBUF
            if b == 0:
                _fire_gather(tab_hbm, idx_all, rows[nb], gsem[nb], l + DEPTH)
            else:
                @pl.when(i < NSTEP - 1)
                def _():
                    _fire_gather(tab_hbm, idx_all, rows[nb], gsem[nb],
                                 l + DEPTH)
        return carry

    lax.fori_loop(0, NSTEP, step, 0)

    for b in range(NBUF):
        _wait_wb(out_hbm, trows[b], wsem[b], wid)


VOCAB = 1000000
RM_P = 512
RM_ROWS = VOCAB * F // 128


def _relayout_body(in_ref, out_ref):
    t = in_ref[...].T.reshape(RM_P, 4, F)
    out_ref[...] = jnp.concatenate([t[:, g, :] for g in range(4)], axis=1)


def _relayout_table(tab_t):
    grid = (RM_ROWS + RM_P - 1) // RM_P
    return pl.pallas_call(
        _relayout_body,
        grid=(grid,),
        in_specs=[pl.BlockSpec((F, 4 * RM_P), lambda i: (0, i))],
        out_specs=pl.BlockSpec((RM_P, 128), lambda i: (i, 0)),
        out_shape=jax.ShapeDtypeStruct((RM_ROWS, 128), jnp.float32),
        compiler_params=pltpu.CompilerParams(
            dimension_semantics=("arbitrary",)),
    )(tab_t)


def kernel(x, table):
    x2 = jnp.transpose(x.astype(jnp.int32)[:, :, 0], (1, 0))
    tab_rm = _relayout_table(table.T).reshape(VOCAB, F)
    out5 = _embed(x2, tab_rm)
    return out5.transpose(2, 4, 0, 1, 3).reshape(BATCH, HIST, F)


# speedup vs baseline: 1.5361x; 1.1459x over previous
import functools

import jax
import jax.numpy as jnp
from jax import lax
from jax.experimental import pallas as pl
from jax.experimental.pallas import tpu as pltpu
from jax.experimental.pallas import tpu_sc as plsc

BATCH = 4096
HIST = 200
F = 32
NC = 2
NS = 16
NW = NC * NS
BB = BATCH // NW
FH = F // 8
NBUF = 4
DEPTH = NBUF - 1
NSTEP = HIST // NBUF


def _fire_gather(tab_hbm, idx_all, rows, gsem, l):
    pltpu.async_copy(tab_hbm.at[idx_all.at[l]], rows, gsem)


def _wait_gather(tab_hbm, idx_all, rows, gsem):
    pltpu.make_async_copy(tab_hbm.at[idx_all.at[0]], rows, gsem).wait()


def _fire_wb(out_hbm, trows, wsem, l, w):
    for fh in range(FH):
        pltpu.async_copy(
            trows.at[pl.ds(fh * 8, 8), pl.ds(0, BB)],
            out_hbm.at[l, fh, w], wsem)


def _wait_wb(out_hbm, trows, wsem, w):
    for fh in range(FH):
        pltpu.make_async_copy(
            trows.at[pl.ds(fh * 8, 8), pl.ds(0, BB)], out_hbm.at[0, fh, w],
            wsem).wait()


def _mask_chunk(idx_all, rows, l):
    acc = jnp.zeros((16,), jnp.int32)
    for g in range(BB // 16):
        v = idx_all[l, pl.ds(g * 16, 16)]
        acc = acc + jnp.where(v <= 0, 1, 0).astype(jnp.int32)
    nz = plsc.all_reduce_population_count(acc > 0)[0]

    @pl.when(nz > 0)
    def _():
        zeros = jnp.zeros((16,), jnp.float32)

        def fix(g, carry):
            off = pl.multiple_of(g * 16, 16)
            v = idx_all[l, pl.ds(off, 16)]
            m = v <= 0
            rid = lax.iota(jnp.int32, 16) + g * 16
            for c in range(F):
                plsc.store_scatter(
                    rows, [rid, jnp.full((16,), c, jnp.int32)],
                    zeros, mask=m)
            return carry

        lax.fori_loop(0, BB // 16, fix, 0)


def _transpose_chunk(rows, trows, iota16):
    cvec = (iota16, iota16 + 16)

    @plsc.parallel_loop(0, BB, 16, unroll=2)
    def _(b0):
        for j in range(16):
            col = jnp.full((16,), 0, jnp.int32) + (b0 + j)
            for h in range(2):
                v = rows[b0 + j, pl.ds(h * 16, 16)]
                plsc.store_scatter(trows, [cvec[h], col], v)


@functools.partial(
    pl.kernel,
    out_type=jax.ShapeDtypeStruct((HIST, FH, NW, 8, BB), jnp.float32),
    mesh=plsc.VectorSubcoreMesh(core_axis_name="c", subcore_axis_name="s"),
    compiler_params=pltpu.CompilerParams(
        needs_layout_passes=False, use_tc_tiling_on_sc=False),
    scratch_types=[
        pltpu.VMEM((HIST, BB), jnp.int32),
        [pltpu.VMEM((BB, F), jnp.float32) for _ in range(NBUF)],
        [pltpu.VMEM((F, BB + 1), jnp.float32) for _ in range(NBUF)],
        [pltpu.SemaphoreType.DMA for _ in range(NBUF)],
        [pltpu.SemaphoreType.DMA for _ in range(NBUF)],
    ],
)
def _embed(x_hbm, tab_hbm, out_hbm, idx_all, rows, trows, gsem, wsem):
    wid = lax.axis_index("s") * NC + lax.axis_index("c")
    iota16 = lax.iota(jnp.int32, 16)

    pltpu.sync_copy(x_hbm.at[:, pl.ds(wid * BB, BB)], idx_all)

    for l in range(DEPTH):
        _fire_gather(tab_hbm, idx_all, rows[l], gsem[l], l)

    def step(i, carry):
        for b in range(NBUF):
            l = i * NBUF + b
            _wait_gather(tab_hbm, idx_all, rows[b], gsem[b])
            _mask_chunk(idx_all, rows[b], l)
            @pl.when(i > 0)
            def _():
                _wait_wb(out_hbm, trows[b], wsem[b], wid)
            _transpose_chunk(rows[b], trows[b], iota16)
            _fire_wb(out_hbm, trows[b], wsem[b], l, wid)
            nb = (b + DEPTH) % NBUF
            if b == 0:
                _fire_gather(tab_hbm, idx_all, rows[nb], gsem[nb], l + DEPTH)
            else:
                @pl.when(i < NSTEP - 1)
                def _():
                    _fire_gather(tab_hbm, idx_all, rows[nb], gsem[nb],
                                 l + DEPTH)
        return carry

    lax.fori_loop(0, NSTEP, step, 0)

    for b in range(NBUF):
        _wait_wb(out_hbm, trows[b], wsem[b], wid)


VOCAB = 1000000
RM_P = 2048
RM_ROWS = VOCAB * F // 128


def _relayout_body(in_ref, out_ref):
    t = in_ref[...].T.reshape(RM_P, 4, F)
    out_ref[...] = jnp.concatenate([t[:, g, :] for g in range(4)], axis=1)


def _relayout_table(tab_t):
    grid = (RM_ROWS + RM_P - 1) // RM_P
    return pl.pallas_call(
        _relayout_body,
        grid=(grid,),
        in_specs=[pl.BlockSpec((F, 4 * RM_P), lambda i: (0, i))],
        out_specs=pl.BlockSpec((RM_P, 128), lambda i: (i, 0)),
        out_shape=jax.ShapeDtypeStruct((RM_ROWS, 128), jnp.float32),
        compiler_params=pltpu.CompilerParams(
            dimension_semantics=("arbitrary",)),
    )(tab_t)


def kernel(x, table):
    x2 = jnp.transpose(x.astype(jnp.int32)[:, :, 0], (1, 0))
    tab_rm = _relayout_table(table.T).reshape(VOCAB, F)
    out5 = _embed(x2, tab_rm)
    return out5.transpose(2, 4, 0, 1, 3).reshape(BATCH, HIST, F)


# speedup vs baseline: 1.5630x; 1.0175x over previous
import functools

import jax
import jax.numpy as jnp
from jax import lax
from jax.experimental import pallas as pl
from jax.experimental.pallas import tpu as pltpu
from jax.experimental.pallas import tpu_sc as plsc

BATCH = 4096
HIST = 200
F = 32
NC = 2
NS = 16
NW = NC * NS
BB = BATCH // NW
FH = F // 8
NBUF = 4
DEPTH = NBUF - 1
NSTEP = HIST // NBUF


def _fire_gather(tab_hbm, idx_all, rows, gsem, l):
    pltpu.async_copy(tab_hbm.at[idx_all.at[l]], rows, gsem)


def _wait_gather(tab_hbm, idx_all, rows, gsem):
    pltpu.make_async_copy(tab_hbm.at[idx_all.at[0]], rows, gsem).wait()


def _fire_wb(out_hbm, trows, wsem, l, w):
    for fh in range(FH):
        pltpu.async_copy(
            trows.at[pl.ds(fh * 8, 8), pl.ds(0, BB)],
            out_hbm.at[l, fh, w], wsem)


def _wait_wb(out_hbm, trows, wsem, w):
    for fh in range(FH):
        pltpu.make_async_copy(
            trows.at[pl.ds(fh * 8, 8), pl.ds(0, BB)], out_hbm.at[0, fh, w],
            wsem).wait()


def _mask_chunk(idx_all, rows, l):
    acc = jnp.zeros((16,), jnp.int32)
    for g in range(BB // 16):
        v = idx_all[l, pl.ds(g * 16, 16)]
        acc = acc + jnp.where(v <= 0, 1, 0).astype(jnp.int32)
    nz = plsc.all_reduce_population_count(acc > 0)[0]

    @pl.when(nz > 0)
    def _():
        zeros = jnp.zeros((16,), jnp.float32)

        def fix(g, carry):
            off = pl.multiple_of(g * 16, 16)
            v = idx_all[l, pl.ds(off, 16)]
            m = v <= 0
            rid = lax.iota(jnp.int32, 16) + g * 16
            for c in range(F):
                plsc.store_scatter(
                    rows, [rid, jnp.full((16,), c, jnp.int32)],
                    zeros, mask=m)
            return carry

        lax.fori_loop(0, BB // 16, fix, 0)


def _transpose_chunk(rows, trows, iota16):
    cvec = (iota16, iota16 + 16)

    @plsc.parallel_loop(0, BB, 16, unroll=2)
    def _(b0):
        for j in range(16):
            col = jnp.full((16,), 0, jnp.int32) + (b0 + j)
            for h in range(2):
                v = rows[b0 + j, pl.ds(h * 16, 16)]
                plsc.store_scatter(trows, [cvec[h], col], v)


@functools.partial(
    pl.kernel,
    out_type=jax.ShapeDtypeStruct((HIST, FH, NW, 8, BB), jnp.float32),
    mesh=plsc.VectorSubcoreMesh(core_axis_name="c", subcore_axis_name="s"),
    compiler_params=pltpu.CompilerParams(
        needs_layout_passes=False, use_tc_tiling_on_sc=False),
    scratch_types=[
        pltpu.VMEM((HIST, BB), jnp.int32),
        [pltpu.VMEM((BB, F), jnp.float32) for _ in range(NBUF)],
        [pltpu.VMEM((F, BB + 1), jnp.float32) for _ in range(NBUF)],
        [pltpu.SemaphoreType.DMA for _ in range(NBUF)],
        [pltpu.SemaphoreType.DMA for _ in range(NBUF)],
    ],
)
def _embed(x_hbm, tab_hbm, out_hbm, idx_all, rows, trows, gsem, wsem):
    wid = lax.axis_index("s") * NC + lax.axis_index("c")
    iota16 = lax.iota(jnp.int32, 16)

    pltpu.sync_copy(x_hbm.at[:, pl.ds(wid * BB, BB)], idx_all)

    for l in range(DEPTH):
        _fire_gather(tab_hbm, idx_all, rows[l], gsem[l], l)

    def step(i, carry):
        for b in range(NBUF):
            l = i * NBUF + b
            _wait_gather(tab_hbm, idx_all, rows[b], gsem[b])
            _mask_chunk(idx_all, rows[b], l)
            @pl.when(i > 0)
            def _():
                _wait_wb(out_hbm, trows[b], wsem[b], wid)
            _transpose_chunk(rows[b], trows[b], iota16)
            _fire_wb(out_hbm, trows[b], wsem[b], l, wid)
            nb = (b + DEPTH) % NBUF
            if b == 0:
                _fire_gather(tab_hbm, idx_all, rows[nb], gsem[nb], l + DEPTH)
            else:
                @pl.when(i < NSTEP - 1)
                def _():
                    _fire_gather(tab_hbm, idx_all, rows[nb], gsem[nb],
                                 l + DEPTH)
        return carry

    lax.fori_loop(0, NSTEP, step, 0)

    for b in range(NBUF):
        _wait_wb(out_hbm, trows[b], wsem[b], wid)


VOCAB = 1000000
RM_P = 8192
RM_ROWS = VOCAB * F // 128


def _relayout_body(in_ref, out_ref):
    t = in_ref[...].T.reshape(RM_P, 4, F)
    out_ref[...] = jnp.concatenate([t[:, g, :] for g in range(4)], axis=1)


def _relayout_table(tab_t):
    grid = (RM_ROWS + RM_P - 1) // RM_P
    return pl.pallas_call(
        _relayout_body,
        grid=(grid,),
        in_specs=[pl.BlockSpec((F, 4 * RM_P), lambda i: (0, i))],
        out_specs=pl.BlockSpec((RM_P, 128), lambda i: (i, 0)),
        out_shape=jax.ShapeDtypeStruct((RM_ROWS, 128), jnp.float32),
        compiler_params=pltpu.CompilerParams(
            dimension_semantics=("arbitrary",)),
    )(tab_t)


def kernel(x, table):
    x2 = jnp.transpose(x.astype(jnp.int32)[:, :, 0], (1, 0))
    tab_rm = _relayout_table(table.T).reshape(VOCAB, F)
    out5 = _embed(x2, tab_rm)
    return out5.transpose(2, 4, 0, 1, 3).reshape(BATCH, HIST, F)
